# Initial kernel scaffold; baseline (speedup 1.0000x reference)
import jax
import jax.numpy as jnp
from jax.experimental import pallas as pl


def kernel(params, atom_emb_idx, nbr_emb, nbr_adj_list, atom_amino_idx, atom_mask):
    raise NotImplementedError("write your pallas kernel here")


# speedup vs baseline: 6.5594x; 6.5594x over previous
import functools

import jax
import jax.numpy as jnp
from jax import lax
from jax.experimental import pallas as pl
from jax.experimental.pallas import tpu as pltpu
from jax.experimental.pallas import tpu_sc as plsc

_B, _N, _M = 4, 2500, 32
_HA, _HB = 64, 16
_H2 = 2 * _HA
_VOCAB, _HINIT = 100, 92
_NA = 2000
_EPS = 1e-5
_NW = 32
_CH = 80
_A = 400
_NBLK = (_B * _N) // _A
_E = _B * _N * _M
_R1 = float(_E)
_R2 = float(_B * _N)

_pcall = pl.pallas_call


def _sc_gather(p_rows, gidx):
    per_w = _E // _NW
    n_ch = per_w // _CH
    mesh = plsc.VectorSubcoreMesh(core_axis_name="c", subcore_axis_name="s")

    @functools.partial(
        pl.kernel,
        mesh=mesh,
        out_type=jax.ShapeDtypeStruct((_E, _H2), jnp.float32),
        scratch_types=[
            pltpu.VMEM((per_w,), jnp.int32),
            pltpu.VMEM((_CH, _H2), jnp.float32),
            pltpu.SemaphoreType.DMA,
        ],
    )
    def gk(p_hbm, idx_hbm, out_hbm, idx_v, rows_v, sem):
        wid = lax.axis_index("s") * 2 + lax.axis_index("c")
        base = pl.multiple_of(wid * per_w, per_w)
        pltpu.sync_copy(idx_hbm.at[pl.ds(base, per_w)], idx_v)

        def body(c, carry):
            off = pl.multiple_of(c * _CH, _CH)
            pltpu.async_copy(
                p_hbm.at[idx_v.at[pl.ds(off, _CH)]], rows_v, sem
            ).wait()
            pltpu.sync_copy(rows_v, out_hbm.at[pl.ds(base + off, _CH)])
            return carry

        lax.fori_loop(0, n_ch, body, 0)

    return gk(p_rows, gidx)


def _sc_segsum(ae_wide, amino_flat):
    n_ch = amino_flat.shape[0] // _CH
    zeros = jnp.zeros((_NA, _H2), jnp.float32)
    mesh = plsc.VectorSubcoreMesh(core_axis_name="c", subcore_axis_name="s")

    out_ch = _NA // _CH

    @functools.partial(
        pl.kernel,
        mesh=mesh,
        out_type=jax.ShapeDtypeStruct((2 * _NA, _H2), jnp.float32),
        scratch_types=[
            pltpu.VMEM((_CH,), jnp.int32),
            pltpu.VMEM((_CH, _H2), jnp.float32),
            pltpu.VMEM_SHARED((_NA, _H2), jnp.float32),
        ],
    )
    def sk(ae_hbm, idx_hbm, z_hbm, out_hbm, idx_v, rows_v, shared):
        cid = lax.axis_index("c")
        sid = lax.axis_index("s")
        wid = sid * 2 + cid

        @pl.when(sid == 0)
        def _():
            pltpu.sync_copy(z_hbm, shared)

        plsc.subcore_barrier()

        for t in range((n_ch + _NW - 1) // _NW):
            j = wid + _NW * t

            @pl.when(j < n_ch)
            def _(j=j):
                row_off = pl.multiple_of(j * _CH, _CH)
                pltpu.sync_copy(idx_hbm.at[pl.ds(row_off, _CH)], idx_v)
                pltpu.sync_copy(ae_hbm.at[pl.ds(row_off, _CH)], rows_v)
                pltpu.sync_copy(rows_v, shared.at[idx_v], add=True)

        plsc.subcore_barrier()

        for t in range((out_ch + 15) // 16):
            j = sid + 16 * t

            @pl.when(j < out_ch)
            def _(j=j):
                o = pl.multiple_of(j * _CH, _CH)
                od = pl.multiple_of(cid * _NA + j * _CH, _CH)
                pltpu.sync_copy(shared.at[pl.ds(o, _CH)],
                                out_hbm.at[pl.ds(od, _CH)])

    return sk(ae_wide, amino_flat, zeros)


def _dot(a, b):
    return jnp.dot(a, b, preferred_element_type=jnp.float32)


def _embed_body(idx_ref, table_ref, we_ref, be_ref, wn_ref, ae_ref, p_ref):
    t = _dot(table_ref[...], we_ref[...])
    idx = idx_ref[...]
    col = lax.broadcasted_iota(jnp.int32, (idx.shape[0], _VOCAB), 1)
    onehot = (idx == col).astype(jnp.float32)
    ae = _dot(onehot, t) + be_ref[...]
    ae_ref[...] = ae
    p_ref[...] = _dot(ae, wn_ref[...])


def _embed(idx, table, we, be, wn0):
    blk = 2000
    return _pcall(
        _embed_body,
        grid=(_B * _N // blk,),
        in_specs=[
            pl.BlockSpec((blk, 1), lambda i: (i, 0)),
            pl.BlockSpec((_VOCAB, _HINIT), lambda i: (0, 0)),
            pl.BlockSpec((_HINIT, _HA), lambda i: (0, 0)),
            pl.BlockSpec((1, _HA), lambda i: (0, 0)),
            pl.BlockSpec((_HA, _H2), lambda i: (0, 0)),
        ],
        out_specs=[
            pl.BlockSpec((blk, _HA), lambda i: (i, 0)),
            pl.BlockSpec((blk, _H2), lambda i: (i, 0)),
        ],
        out_shape=[
            jax.ShapeDtypeStruct((_B * _N, _HA), jnp.float32),
            jax.ShapeDtypeStruct((_B * _N, _H2), jnp.float32),
        ],
    )(idx, table, we, be, wn0)


def _gated_half(ae, gh, ne, wc, we, b):
    pc = _dot(ae, wc)
    pcb = jnp.broadcast_to(pc[:, None, :], (_A, _M, _HA)).reshape(_A * _M, _HA)
    return pcb + gh + _dot(ne, we) + b


def _pass1_body(ae_ref, g_ref, ne_ref, wcf, wcc, wef, wec, bf, bc,
                sum_f, ssq_f, sum_c, ssq_c):
    ae = ae_ref[...]
    ne = ne_ref[...]
    g = g_ref[...]
    yf = _gated_half(ae, g[:, :_HA], ne, wcf[...], wef[...], bf[...])
    yc = _gated_half(ae, g[:, _HA:], ne, wcc[...], wec[...], bc[...])

    @pl.when(pl.program_id(0) == 0)
    def _():
        sum_f[...] = jnp.zeros_like(sum_f)
        ssq_f[...] = jnp.zeros_like(ssq_f)
        sum_c[...] = jnp.zeros_like(sum_c)
        ssq_c[...] = jnp.zeros_like(ssq_c)

    sum_f[...] += jnp.sum(yf, axis=0, keepdims=True)
    ssq_f[...] += jnp.sum(yf * yf, axis=0, keepdims=True)
    sum_c[...] += jnp.sum(yc, axis=0, keepdims=True)
    ssq_c[...] += jnp.sum(yc * yc, axis=0, keepdims=True)


def _affine(s, ss, gamma, beta, r):
    mu = s * (1.0 / r)
    var = ss * (1.0 / r) - mu * mu
    scale = gamma * lax.rsqrt(var + _EPS)
    return scale, beta - mu * scale


def _pass2_body(ae_ref, g_ref, ne_ref, wcf, wcc, wef, wec, bf, bc,
                sf, qf, sc_, qc, ghf, bhf, ghc, bhc,
                ns_ref, sum2, ssq2):
    ae = ae_ref[...]
    ne = ne_ref[...]
    g = g_ref[...]
    yf = _gated_half(ae, g[:, :_HA], ne, wcf[...], wef[...], bf[...])
    yc = _gated_half(ae, g[:, _HA:], ne, wcc[...], wec[...], bc[...])
    scale_f, shift_f = _affine(sf[...], qf[...], ghf[...], bhf[...], _R1)
    scale_c, shift_c = _affine(sc_[...], qc[...], ghc[...], bhc[...], _R1)
    filt = jax.nn.sigmoid(yf * scale_f + shift_f)
    core = jnp.maximum(yc * scale_c + shift_c, 0.0)
    ns = jnp.sum((filt * core).reshape(_A, _M, _HA), axis=1)
    ns_ref[...] = ns

    @pl.when(pl.program_id(0) == 0)
    def _():
        sum2[...] = jnp.zeros_like(sum2)
        ssq2[...] = jnp.zeros_like(ssq2)

    sum2[...] += jnp.sum(ns, axis=0, keepdims=True)
    ssq2[...] += jnp.sum(ns * ns, axis=0, keepdims=True)


def _conv_passes(ae, g, ne, wsplit, bn_h_g, bn_h_b):
    wcf, wcc, wef, wec, bf, bc = wsplit
    row = lambda i: (i, 0)
    full = lambda i: (0, 0)
    w64 = pl.BlockSpec((_HA, _HA), full)
    w16 = pl.BlockSpec((_HB, _HA), full)
    v64 = pl.BlockSpec((1, _HA), full)
    data_specs = [
        pl.BlockSpec((_A, _HA), row),
        pl.BlockSpec((_A * _M, _H2), row),
        pl.BlockSpec((_A * _M, _HB), row),
        w64, w64, w16, w16, v64, v64,
    ]
    stat = jax.ShapeDtypeStruct((1, _HA), jnp.float32)
    sf, qf, sc_, qc = _pcall(
        _pass1_body,
        grid=(_NBLK,),
        in_specs=data_specs,
        out_specs=[pl.BlockSpec((1, _HA), full)] * 4,
        out_shape=[stat] * 4,
    )(ae, g, ne, wcf, wcc, wef, wec, bf, bc)

    ghf, ghc = bn_h_g[:_HA].reshape(1, _HA), bn_h_g[_HA:].reshape(1, _HA)
    bhf, bhc = bn_h_b[:_HA].reshape(1, _HA), bn_h_b[_HA:].reshape(1, _HA)
    ns, s2, q2 = _pcall(
        _pass2_body,
        grid=(_NBLK,),
        in_specs=data_specs + [v64] * 8,
        out_specs=[
            pl.BlockSpec((_A, _HA), row),
            pl.BlockSpec((1, _HA), full),
            pl.BlockSpec((1, _HA), full),
        ],
        out_shape=[
            jax.ShapeDtypeStruct((_B * _N, _HA), jnp.float32),
            stat, stat,
        ],
    )(ae, g, ne, wcf, wcc, wef, wec, bf, bc,
      sf, qf, sc_, qc, ghf, bhf, ghc, bhc)
    return ns, s2, q2


def _pass3_body(ae_ref, ns_ref, s2, q2, g2, b2, wn_ref, out_ref, p_ref):
    scale, shift = _affine(s2[...], q2[...], g2[...], b2[...], _R2)
    res = jnp.maximum(ae_ref[...] + ns_ref[...] * scale + shift, 0.0)
    out_ref[...] = res
    p_ref[...] = _dot(res, wn_ref[...])


def _pass3(ae, ns, s2, q2, g2, b2, wn_next):
    return _pcall(
        _pass3_body,
        out_shape=[
            jax.ShapeDtypeStruct((_B * _N, _HA), jnp.float32),
            jax.ShapeDtypeStruct((_B * _N, _H2), jnp.float32),
        ],
    )(ae, ns, s2, q2, g2.reshape(1, _HA), b2.reshape(1, _HA), wn_next)


def _pass3_last_body(ae_ref, ns_ref, s2, q2, g2, b2, out_ref, sum_ref):
    scale, shift = _affine(s2[...], q2[...], g2[...], b2[...], _R2)
    res = jnp.maximum(ae_ref[...] + ns_ref[...] * scale + shift, 0.0)
    out_ref[...] = jnp.concatenate([res, jnp.zeros_like(res)], axis=1)
    sum_ref[...] = jnp.sum(res.reshape(_B, _N, _HA), axis=1)


def _pass3_last(ae, ns, s2, q2, g2, b2):
    return _pcall(
        _pass3_last_body,
        out_shape=[
            jax.ShapeDtypeStruct((_B * _N, _H2), jnp.float32),
            jax.ShapeDtypeStruct((_B, _HA), jnp.float32),
        ],
    )(ae, ns, s2.reshape(1, _HA), q2.reshape(1, _HA),
      g2.reshape(1, _HA), b2.reshape(1, _HA))


def _head_body(pooled_ref, sum_ref, idx0_ref, wa, ba, wao, bao, wc, bc_, wo, bo,
               prot_ref, amino_ref, mask_ref):
    protein = jnp.maximum(sum_ref[...] * (1.0 / _N), 0.0)
    protein = jnp.maximum(_dot(protein, wc[...]) + bc_[...], 0.0)
    prot_ref[...] = _dot(protein, wo[...]) + bo[...]
    pooled2 = pooled_ref[...]
    pooled = pooled2[:_NA, :_HA] + pooled2[_NA:, :_HA]
    am = jnp.maximum(pooled, 0.0)
    am = jnp.maximum(_dot(am, wa[...]) + ba[...], 0.0)
    amino_ref[...] = _dot(am, wao[...]) + bao[...]
    row = lax.broadcasted_iota(jnp.int32, (_NA, 1), 0)
    mask_ref[...] = (row >= idx0_ref[...]).astype(jnp.int32)


def _head(pooled, summed, idx0, wa, ba, wao, bao, wc, bc_, wo, bo):
    fb = lambda shape: pl.BlockSpec(shape, lambda: (0, 0))
    return _pcall(
        _head_body,
        in_specs=[
            fb((2 * _NA, _H2)),
            fb((_B, _HA)), fb((1, 1)),
            fb((_HA, 32)), fb((1, 32)), fb((32, 1)), fb((1, 1)),
            fb((_HA, 32)), fb((1, 32)), fb((32, 1)), fb((1, 1)),
        ],
        out_shape=[
            jax.ShapeDtypeStruct((_B, 1), jnp.float32),
            jax.ShapeDtypeStruct((_NA, 1), jnp.float32),
            jax.ShapeDtypeStruct((_NA, 1), jnp.int32),
        ],
    )(pooled, summed, idx0, wa, ba, wao, bao, wc, bc_, wo, bo)


def kernel(params, atom_emb_idx, nbr_emb, nbr_adj_list, atom_amino_idx,
           atom_mask):
    p = params
    convs = p["convs"]
    we, be = p["embed"]
    idx2d = atom_emb_idx.reshape(_B * _N, 1).astype(jnp.int32)
    wn = [c["W"][_HA:2 * _HA, :] for c in convs]
    ae, prows = _embed(idx2d, p["atom_table"], we, be.reshape(1, _HA), wn[0])

    offs = (jnp.arange(_B, dtype=jnp.int32) * _N)[:, None, None]
    gidx = (nbr_adj_list.astype(jnp.int32) + offs).reshape(_E)
    ne_flat = nbr_emb.reshape(_E, _HB)

    for li, c in enumerate(convs):
        w, b = c["W"], c["b"]
        wsplit = (
            w[0:_HA, 0:_HA], w[0:_HA, _HA:_H2],
            w[2 * _HA:, 0:_HA], w[2 * _HA:, _HA:_H2],
            b[0:_HA].reshape(1, _HA), b[_HA:].reshape(1, _HA),
        )
        g = _sc_gather(prows, gidx)
        ns, s2, q2 = _conv_passes(ae, g, ne_flat, wsplit,
                                  c["bn_h_g"], c["bn_h_b"])
        if li + 1 < len(convs):
            ae, prows = _pass3(ae, ns, s2, q2, c["bn_o_g"], c["bn_o_b"],
                               wn[li + 1])
        else:
            ae_wide, summed = _pass3_last(ae, ns, s2, q2,
                                          c["bn_o_g"], c["bn_o_b"])

    pooled = _sc_segsum(ae_wide, atom_amino_idx.reshape(-1).astype(jnp.int32))

    wa, ba = p["amino_to_fc"]
    wao, bao = p["fc_amino_out"]
    wc, bc_ = p["conv_to_fc"]
    wo, bo = p["fc_out"]
    idx0 = atom_amino_idx.reshape(1, -1)[0:1, 0:1].astype(jnp.int32)
    prot, amino, mask_i = _head(
        pooled, summed, idx0,
        wa, ba.reshape(1, -1), wao, bao.reshape(1, -1),
        wc, bc_.reshape(1, -1), wo, bo.reshape(1, -1),
    )
    return (prot, amino, mask_i.astype(jnp.uint8))


# speedup vs baseline: 9.0482x; 1.3794x over previous
import functools

import jax
import jax.numpy as jnp
from jax import lax
from jax.experimental import pallas as pl
from jax.experimental.pallas import tpu as pltpu
from jax.experimental.pallas import tpu_sc as plsc

_B, _N, _M = 4, 2500, 32
_HA, _HB = 64, 16
_H2 = 2 * _HA
_VOCAB, _HINIT = 100, 92
_NA = 2000
_EPS = 1e-5
_NW = 32
_CH = 80
_A = 400
_NBLK = (_B * _N) // _A
_E = _B * _N * _M
_R1 = float(_E)
_R2 = float(_B * _N)

_pcall = pl.pallas_call


def _sc_gather(p_rows, gidx):
    nrows = p_rows.shape[0]
    per_w = _E // _NW
    n_ch = per_w // _CH
    stage = 1000
    n_stage = nrows // stage
    mesh = plsc.VectorSubcoreMesh(core_axis_name="c", subcore_axis_name="s")

    @functools.partial(
        pl.kernel,
        mesh=mesh,
        out_type=jax.ShapeDtypeStruct((_E, _H2), jnp.float32),
        scratch_types=[
            pltpu.VMEM((per_w,), jnp.int32),
            pltpu.VMEM((_CH, _H2), jnp.float32),
            pltpu.VMEM((_CH, _H2), jnp.float32),
            pltpu.VMEM_SHARED((nrows, _H2), jnp.float32),
            pltpu.SemaphoreType.DMA,
            pltpu.SemaphoreType.DMA,
        ],
    )
    def gk(p_hbm, idx_hbm, out_hbm, idx_v, rows_a, rows_b, table, sem_a, sem_b):
        cid = lax.axis_index("c")
        sid = lax.axis_index("s")
        wid = sid * 2 + cid
        base = pl.multiple_of(wid * per_w, per_w)

        @pl.when(sid < n_stage)
        def _():
            so = pl.multiple_of(sid * stage, 8)
            pltpu.sync_copy(p_hbm.at[pl.ds(so, stage)],
                            table.at[pl.ds(so, stage)])

        pltpu.sync_copy(idx_hbm.at[pl.ds(base, per_w)], idx_v)
        plsc.subcore_barrier()

        def fire(c, buf, sem):
            off = pl.multiple_of(c * _CH, _CH)
            pltpu.make_async_copy(
                table.at[idx_v.at[pl.ds(off, _CH)]], buf, sem
            ).start()

        def drain_write(c, buf, sem):
            off = pl.multiple_of(c * _CH, _CH)
            pltpu.make_async_copy(
                table.at[idx_v.at[pl.ds(off, _CH)]], buf, sem
            ).wait()
            pltpu.sync_copy(buf, out_hbm.at[pl.ds(base + off, _CH)])

        fire(0, rows_a, sem_a)

        def body(i, carry):
            c0 = i * 2
            c1 = c0 + 1
            c2 = c0 + 2

            @pl.when(c1 < n_ch)
            def _():
                fire(c1, rows_b, sem_b)

            drain_write(c0, rows_a, sem_a)

            @pl.when(c1 < n_ch)
            def _():
                @pl.when(c2 < n_ch)
                def _():
                    fire(c2, rows_a, sem_a)

                drain_write(c1, rows_b, sem_b)

            return carry

        lax.fori_loop(0, (n_ch + 1) // 2, body, 0)

    return gk(p_rows, gidx)


def _sc_segsum(ae_wide, amino_flat):
    n_ch = amino_flat.shape[0] // _CH
    zeros = jnp.zeros((_NA, _H2), jnp.float32)
    mesh = plsc.VectorSubcoreMesh(core_axis_name="c", subcore_axis_name="s")

    out_ch = _NA // _CH

    @functools.partial(
        pl.kernel,
        mesh=mesh,
        out_type=jax.ShapeDtypeStruct((2 * _NA, _H2), jnp.float32),
        scratch_types=[
            pltpu.VMEM((_CH,), jnp.int32),
            pltpu.VMEM((_CH, _H2), jnp.float32),
            pltpu.VMEM_SHARED((_NA, _H2), jnp.float32),
        ],
    )
    def sk(ae_hbm, idx_hbm, z_hbm, out_hbm, idx_v, rows_v, shared):
        cid = lax.axis_index("c")
        sid = lax.axis_index("s")
        wid = sid * 2 + cid

        @pl.when(sid == 0)
        def _():
            pltpu.sync_copy(z_hbm, shared)

        plsc.subcore_barrier()

        for t in range((n_ch + _NW - 1) // _NW):
            j = wid + _NW * t

            @pl.when(j < n_ch)
            def _(j=j):
                row_off = pl.multiple_of(j * _CH, _CH)
                pltpu.sync_copy(idx_hbm.at[pl.ds(row_off, _CH)], idx_v)
                pltpu.sync_copy(ae_hbm.at[pl.ds(row_off, _CH)], rows_v)
                pltpu.sync_copy(rows_v, shared.at[idx_v], add=True)

        plsc.subcore_barrier()

        for t in range((out_ch + 15) // 16):
            j = sid + 16 * t

            @pl.when(j < out_ch)
            def _(j=j):
                o = pl.multiple_of(j * _CH, _CH)
                od = pl.multiple_of(cid * _NA + j * _CH, _CH)
                pltpu.sync_copy(shared.at[pl.ds(o, _CH)],
                                out_hbm.at[pl.ds(od, _CH)])

    return sk(ae_wide, amino_flat, zeros)


def _dot(a, b):
    return jnp.dot(a, b, preferred_element_type=jnp.float32)


def _embed_body(idx_ref, table_ref, we_ref, be_ref, wn_ref, ae_ref, p_ref):
    t = _dot(table_ref[...], we_ref[...])
    idx = idx_ref[...]
    col = lax.broadcasted_iota(jnp.int32, (idx.shape[0], _VOCAB), 1)
    onehot = (idx == col).astype(jnp.float32)
    ae = _dot(onehot, t) + be_ref[...]
    ae_ref[...] = ae
    p_ref[...] = _dot(ae, wn_ref[...])


def _embed(idx, table, we, be, wn0):
    blk = 2000
    return _pcall(
        _embed_body,
        grid=(_B * _N // blk,),
        in_specs=[
            pl.BlockSpec((blk, 1), lambda i: (i, 0)),
            pl.BlockSpec((_VOCAB, _HINIT), lambda i: (0, 0)),
            pl.BlockSpec((_HINIT, _HA), lambda i: (0, 0)),
            pl.BlockSpec((1, _HA), lambda i: (0, 0)),
            pl.BlockSpec((_HA, _H2), lambda i: (0, 0)),
        ],
        out_specs=[
            pl.BlockSpec((blk, _HA), lambda i: (i, 0)),
            pl.BlockSpec((blk, _H2), lambda i: (i, 0)),
        ],
        out_shape=[
            jax.ShapeDtypeStruct((_B * _N, _HA), jnp.float32),
            jax.ShapeDtypeStruct((_B * _N, _H2), jnp.float32),
        ],
    )(idx, table, we, be, wn0)


def _gated_half(ae, gh, ne, wc, we, b):
    pc = _dot(ae, wc)
    pcb = jnp.broadcast_to(pc[:, None, :], (_A, _M, _HA)).reshape(_A * _M, _HA)
    return pcb + gh + _dot(ne, we) + b


def _pass1_body(ae_ref, g_ref, ne_ref, wcf, wcc, wef, wec, bf, bc,
                sum_f, ssq_f, sum_c, ssq_c):
    ae = ae_ref[...]
    ne = ne_ref[...]
    g = g_ref[...]
    yf = _gated_half(ae, g[:, :_HA], ne, wcf[...], wef[...], bf[...])
    yc = _gated_half(ae, g[:, _HA:], ne, wcc[...], wec[...], bc[...])

    @pl.when(pl.program_id(0) == 0)
    def _():
        sum_f[...] = jnp.zeros_like(sum_f)
        ssq_f[...] = jnp.zeros_like(ssq_f)
        sum_c[...] = jnp.zeros_like(sum_c)
        ssq_c[...] = jnp.zeros_like(ssq_c)

    sum_f[...] += jnp.sum(yf, axis=0, keepdims=True)
    ssq_f[...] += jnp.sum(yf * yf, axis=0, keepdims=True)
    sum_c[...] += jnp.sum(yc, axis=0, keepdims=True)
    ssq_c[...] += jnp.sum(yc * yc, axis=0, keepdims=True)


def _affine(s, ss, gamma, beta, r):
    mu = s * (1.0 / r)
    var = ss * (1.0 / r) - mu * mu
    scale = gamma * lax.rsqrt(var + _EPS)
    return scale, beta - mu * scale


def _pass2_body(ae_ref, g_ref, ne_ref, wcf, wcc, wef, wec, bf, bc,
                sf, qf, sc_, qc, ghf, bhf, ghc, bhc,
                ns_ref, sum2, ssq2):
    ae = ae_ref[...]
    ne = ne_ref[...]
    g = g_ref[...]
    yf = _gated_half(ae, g[:, :_HA], ne, wcf[...], wef[...], bf[...])
    yc = _gated_half(ae, g[:, _HA:], ne, wcc[...], wec[...], bc[...])
    scale_f, shift_f = _affine(sf[...], qf[...], ghf[...], bhf[...], _R1)
    scale_c, shift_c = _affine(sc_[...], qc[...], ghc[...], bhc[...], _R1)
    filt = jax.nn.sigmoid(yf * scale_f + shift_f)
    core = jnp.maximum(yc * scale_c + shift_c, 0.0)
    ns = jnp.sum((filt * core).reshape(_A, _M, _HA), axis=1)
    ns_ref[...] = ns

    @pl.when(pl.program_id(0) == 0)
    def _():
        sum2[...] = jnp.zeros_like(sum2)
        ssq2[...] = jnp.zeros_like(ssq2)

    sum2[...] += jnp.sum(ns, axis=0, keepdims=True)
    ssq2[...] += jnp.sum(ns * ns, axis=0, keepdims=True)


def _conv_passes(ae, g, ne, wsplit, bn_h_g, bn_h_b):
    wcf, wcc, wef, wec, bf, bc = wsplit
    row = lambda i: (i, 0)
    full = lambda i: (0, 0)
    w64 = pl.BlockSpec((_HA, _HA), full)
    w16 = pl.BlockSpec((_HB, _HA), full)
    v64 = pl.BlockSpec((1, _HA), full)
    data_specs = [
        pl.BlockSpec((_A, _HA), row),
        pl.BlockSpec((_A * _M, _H2), row),
        pl.BlockSpec((_A * _M, _HB), row),
        w64, w64, w16, w16, v64, v64,
    ]
    stat = jax.ShapeDtypeStruct((1, _HA), jnp.float32)
    sf, qf, sc_, qc = _pcall(
        _pass1_body,
        grid=(_NBLK,),
        in_specs=data_specs,
        out_specs=[pl.BlockSpec((1, _HA), full)] * 4,
        out_shape=[stat] * 4,
    )(ae, g, ne, wcf, wcc, wef, wec, bf, bc)

    ghf, ghc = bn_h_g[:_HA].reshape(1, _HA), bn_h_g[_HA:].reshape(1, _HA)
    bhf, bhc = bn_h_b[:_HA].reshape(1, _HA), bn_h_b[_HA:].reshape(1, _HA)
    ns, s2, q2 = _pcall(
        _pass2_body,
        grid=(_NBLK,),
        in_specs=data_specs + [v64] * 8,
        out_specs=[
            pl.BlockSpec((_A, _HA), row),
            pl.BlockSpec((1, _HA), full),
            pl.BlockSpec((1, _HA), full),
        ],
        out_shape=[
            jax.ShapeDtypeStruct((_B * _N, _HA), jnp.float32),
            stat, stat,
        ],
    )(ae, g, ne, wcf, wcc, wef, wec, bf, bc,
      sf, qf, sc_, qc, ghf, bhf, ghc, bhc)
    return ns, s2, q2


def _pass3_body(ae_ref, ns_ref, s2, q2, g2, b2, wn_ref, out_ref, p_ref):
    scale, shift = _affine(s2[...], q2[...], g2[...], b2[...], _R2)
    res = jnp.maximum(ae_ref[...] + ns_ref[...] * scale + shift, 0.0)
    out_ref[...] = res
    p_ref[...] = _dot(res, wn_ref[...])


def _pass3(ae, ns, s2, q2, g2, b2, wn_next):
    return _pcall(
        _pass3_body,
        out_shape=[
            jax.ShapeDtypeStruct((_B * _N, _HA), jnp.float32),
            jax.ShapeDtypeStruct((_B * _N, _H2), jnp.float32),
        ],
    )(ae, ns, s2, q2, g2.reshape(1, _HA), b2.reshape(1, _HA), wn_next)


def _pass3_last_body(ae_ref, ns_ref, s2, q2, g2, b2, out_ref, sum_ref):
    scale, shift = _affine(s2[...], q2[...], g2[...], b2[...], _R2)
    res = jnp.maximum(ae_ref[...] + ns_ref[...] * scale + shift, 0.0)
    out_ref[...] = jnp.concatenate([res, jnp.zeros_like(res)], axis=1)
    sum_ref[...] = jnp.sum(res.reshape(_B, _N, _HA), axis=1)


def _pass3_last(ae, ns, s2, q2, g2, b2):
    return _pcall(
        _pass3_last_body,
        out_shape=[
            jax.ShapeDtypeStruct((_B * _N, _H2), jnp.float32),
            jax.ShapeDtypeStruct((_B, _HA), jnp.float32),
        ],
    )(ae, ns, s2.reshape(1, _HA), q2.reshape(1, _HA),
      g2.reshape(1, _HA), b2.reshape(1, _HA))


def _head_body(pooled_ref, sum_ref, idx0_ref, wa, ba, wao, bao, wc, bc_, wo, bo,
               prot_ref, amino_ref, mask_ref):
    protein = jnp.maximum(sum_ref[...] * (1.0 / _N), 0.0)
    protein = jnp.maximum(_dot(protein, wc[...]) + bc_[...], 0.0)
    prot_ref[...] = _dot(protein, wo[...]) + bo[...]
    pooled2 = pooled_ref[...]
    pooled = pooled2[:_NA, :_HA] + pooled2[_NA:, :_HA]
    am = jnp.maximum(pooled, 0.0)
    am = jnp.maximum(_dot(am, wa[...]) + ba[...], 0.0)
    amino_ref[...] = _dot(am, wao[...]) + bao[...]
    row = lax.broadcasted_iota(jnp.int32, (_NA, 1), 0)
    mask_ref[...] = (row >= idx0_ref[...]).astype(jnp.int32)


def _head(pooled, summed, idx0, wa, ba, wao, bao, wc, bc_, wo, bo):
    fb = lambda shape: pl.BlockSpec(shape, lambda: (0, 0))
    return _pcall(
        _head_body,
        in_specs=[
            fb((2 * _NA, _H2)),
            fb((_B, _HA)), fb((1, 1)),
            fb((_HA, 32)), fb((1, 32)), fb((32, 1)), fb((1, 1)),
            fb((_HA, 32)), fb((1, 32)), fb((32, 1)), fb((1, 1)),
        ],
        out_shape=[
            jax.ShapeDtypeStruct((_B, 1), jnp.float32),
            jax.ShapeDtypeStruct((_NA, 1), jnp.float32),
            jax.ShapeDtypeStruct((_NA, 1), jnp.int32),
        ],
    )(pooled, summed, idx0, wa, ba, wao, bao, wc, bc_, wo, bo)


def kernel(params, atom_emb_idx, nbr_emb, nbr_adj_list, atom_amino_idx,
           atom_mask):
    p = params
    convs = p["convs"]
    we, be = p["embed"]
    idx2d = atom_emb_idx.reshape(_B * _N, 1).astype(jnp.int32)
    wn = [c["W"][_HA:2 * _HA, :] for c in convs]
    ae, prows = _embed(idx2d, p["atom_table"], we, be.reshape(1, _HA), wn[0])

    offs = (jnp.arange(_B, dtype=jnp.int32) * _N)[:, None, None]
    gidx = (nbr_adj_list.astype(jnp.int32) + offs).reshape(_E)
    ne_flat = nbr_emb.reshape(_E, _HB)

    for li, c in enumerate(convs):
        w, b = c["W"], c["b"]
        wsplit = (
            w[0:_HA, 0:_HA], w[0:_HA, _HA:_H2],
            w[2 * _HA:, 0:_HA], w[2 * _HA:, _HA:_H2],
            b[0:_HA].reshape(1, _HA), b[_HA:].reshape(1, _HA),
        )
        g = _sc_gather(prows, gidx)
        ns, s2, q2 = _conv_passes(ae, g, ne_flat, wsplit,
                                  c["bn_h_g"], c["bn_h_b"])
        if li + 1 < len(convs):
            ae, prows = _pass3(ae, ns, s2, q2, c["bn_o_g"], c["bn_o_b"],
                               wn[li + 1])
        else:
            ae_wide, summed = _pass3_last(ae, ns, s2, q2,
                                          c["bn_o_g"], c["bn_o_b"])

    pooled = _sc_segsum(ae_wide, atom_amino_idx.reshape(-1).astype(jnp.int32))

    wa, ba = p["amino_to_fc"]
    wao, bao = p["fc_amino_out"]
    wc, bc_ = p["conv_to_fc"]
    wo, bo = p["fc_out"]
    idx0 = atom_amino_idx.reshape(1, -1)[0:1, 0:1].astype(jnp.int32)
    prot, amino, mask_i = _head(
        pooled, summed, idx0,
        wa, ba.reshape(1, -1), wao, bao.reshape(1, -1),
        wc, bc_.reshape(1, -1), wo, bo.reshape(1, -1),
    )
    return (prot, amino, mask_i.astype(jnp.uint8))
